# Initial kernel scaffold; baseline (speedup 1.0000x reference)
import jax
import jax.numpy as jnp
from jax.experimental import pallas as pl


def kernel(X_batch, batch_idx, W):
    raise NotImplementedError("write your pallas kernel here")


# speedup vs baseline: 8.0230x; 8.0230x over previous
import jax
import jax.numpy as jnp
from jax.experimental import pallas as pl

N, G, T, S = 320000, 128, 16, 256
SCALE = 1000000.0
BLK = 2560
NB = N // BLK


def _make_seg_kernel(n, g, t, s, blk):
    nb = n // blk

    def body(ids_ref, x_ref, w_ref, ilr_ref, xb_ref):
        i = pl.program_id(0)
        ids = ids_ref[0, 0, :]
        seg = jax.lax.broadcasted_iota(jnp.int32, (s, blk), 0)
        mask = (seg == ids[None, :]).astype(jnp.bfloat16)
        x = x_ref[...].astype(jnp.bfloat16)
        partial = jax.lax.dot_general(
            mask, x, (((1,), (0,)), ((), ())),
            preferred_element_type=jnp.float32)

        @pl.when(i == 0)
        def _init():
            xb_ref[...] = partial

        @pl.when(i > 0)
        def _acc():
            xb_ref[...] += partial

        @pl.when(i == nb - 1)
        def _finish():
            raw = xb_ref[...]
            rs = jnp.sum(raw, axis=1, keepdims=True)
            xb = raw * (SCALE / jnp.clip(rs, 1e-12, None))
            xb_ref[...] = xb
            ilr_ref[...] = jax.lax.dot_general(
                xb, w_ref[...], (((1,), (1,)), ((), ())),
                preferred_element_type=jnp.float32)

    return pl.pallas_call(
        body,
        grid=(nb,),
        in_specs=[
            pl.BlockSpec((1, 1, blk), lambda i: (i, 0, 0)),
            pl.BlockSpec((blk, g), lambda i: (i, 0)),
            pl.BlockSpec((t, g), lambda i: (0, 0)),
        ],
        out_specs=[
            pl.BlockSpec((s, t), lambda i: (0, 0)),
            pl.BlockSpec((s, g), lambda i: (0, 0)),
        ],
        out_shape=[
            jax.ShapeDtypeStruct((s, t), jnp.float32),
            jax.ShapeDtypeStruct((s, g), jnp.float32),
        ],
    )


def kernel(X_batch, batch_idx, W):
    ids3 = batch_idx.astype(jnp.int32).reshape(NB, 1, BLK)
    ilr_y, X_bulk = _make_seg_kernel(N, G, T, S, BLK)(ids3, X_batch, W)
    return (ilr_y, X_bulk)
